# baseline (speedup 1.0000x reference)
import jax
import jax.numpy as jnp
from jax import lax
from jax.experimental import pallas as pl
from jax.experimental.pallas import tpu as pltpu
from jax.experimental.pallas import tpu_sc as plsc

N = 10000
E = 320000
NFEAT = 128
HDIM = 64
NCLASS = 40

NC = 2
NS = 16
NW = NC * NS
E_PER_W = E // NW
CHUNK = 128
NCHUNK = E_PER_W // CHUNK
REM = E_PER_W - NCHUNK * CHUNK
G = 3
NGRP = NCHUNK // G
LAG = 16
NPAD = 10240
ROWS_PER_S = NPAD // NS
BR = 2000

_SC_MESH = plsc.VectorSubcoreMesh(
    core_axis_name="c", subcore_axis_name="s", num_cores=NC, num_subcores=NS)
_SC_PARAMS = pltpu.CompilerParams(use_tc_tiling_on_sc=False)


def _deg_body(ei_hbm, ones_hbm, zeros_hbm, out_hbm, idx_d, ones_v, acc, sem):
  c = lax.axis_index("c")
  s = lax.axis_index("s")
  pltpu.sync_copy(zeros_hbm.at[pl.ds(s * ROWS_PER_S, ROWS_PER_S)],
                  acc.at[pl.ds(s * ROWS_PER_S, ROWS_PER_S)])
  pltpu.sync_copy(ei_hbm.at[pl.ds(E + (c * NS + s) * E_PER_W, E_PER_W)], idx_d)
  pltpu.sync_copy(ones_hbm, ones_v)
  plsc.subcore_barrier()

  def drain():
    pltpu.make_async_copy(ones_hbm, ones_v, sem).wait()

  def step(j, carry):
    pltpu.async_copy(ones_v, acc.at[idx_d.at[pl.ds(j * CHUNK, CHUNK)]],
                     sem, add=True)

    @pl.when(j >= LAG)
    def _():
      drain()

    return carry

  lax.fori_loop(0, NCHUNK, step, 0)
  for _ in range(LAG):
    drain()
  pltpu.sync_copy(ones_v.at[pl.ds(0, REM)],
                  acc.at[idx_d.at[pl.ds(NCHUNK * CHUNK, REM)]], add=True)
  plsc.subcore_barrier()
  pltpu.sync_copy(acc.at[pl.ds(s * ROWS_PER_S, ROWS_PER_S)],
                  out_hbm.at[c, pl.ds(s * ROWS_PER_S, ROWS_PER_S)])


def _sc_degree(ei_flat):
  ones = jnp.ones((CHUNK, 16), jnp.float32)
  zeros = jnp.zeros((NPAD, 16), jnp.float32)
  return pl.kernel(
      _deg_body,
      out_type=jax.ShapeDtypeStruct((NC, NPAD, 16), jnp.float32),
      mesh=_SC_MESH,
      scratch_types=[
          pltpu.VMEM((E_PER_W,), jnp.int32),
          pltpu.VMEM((CHUNK, 16), jnp.float32),
          pltpu.VMEM_SHARED((NPAD, 16), jnp.float32),
          pltpu.SemaphoreType.DMA,
      ],
      compiler_params=_SC_PARAMS,
  )(ei_flat, ones, zeros)


def _edge_body(g_hbm, ei_hbm, zeros_hbm, out_hbm,
               idx_s, idx_d, rows, acc, gsem, ssem):
  c = lax.axis_index("c")
  s = lax.axis_index("s")
  base = (c * NS + s) * E_PER_W
  pltpu.sync_copy(ei_hbm.at[pl.ds(base, E_PER_W)], idx_s)

  def fire_gather(j, slot):
    pltpu.async_copy(g_hbm.at[idx_s.at[pl.ds(j * CHUNK, CHUNK)]],
                     rows.at[slot], gsem)

  def drain_gather(slot):
    pltpu.make_async_copy(g_hbm.at[pl.ds(0, CHUNK)], rows.at[slot], gsem).wait()

  def fire_scatter(j, slot):
    pltpu.async_copy(rows.at[slot], acc.at[idx_d.at[pl.ds(j * CHUNK, CHUNK)]],
                     ssem, add=True)

  def drain_scatter(slot):
    pltpu.make_async_copy(g_hbm.at[pl.ds(0, CHUNK)], rows.at[slot], ssem).wait()

  for b in range(G):
    fire_gather(b, b)
  pltpu.sync_copy(ei_hbm.at[pl.ds(E + base, E_PER_W)], idx_d)
  pltpu.sync_copy(zeros_hbm.at[pl.ds(s * ROWS_PER_S, ROWS_PER_S)],
                  acc.at[pl.ds(s * ROWS_PER_S, ROWS_PER_S)])
  plsc.subcore_barrier()

  def body(J, carry):
    cur = lax.rem(J, 2)
    nxt = 1 - cur

    @pl.when(J >= 1)
    def _():
      for b in range(G):
        drain_scatter(nxt * G + b)

    @pl.when(J + 1 < NGRP)
    def _():
      for b in range(G):
        fire_gather((J + 1) * G + b, nxt * G + b)

    for b in range(G):
      drain_gather(cur * G + b)
    for b in range(G):
      fire_scatter(J * G + b, cur * G + b)
    return carry

  lax.fori_loop(0, NGRP, body, 0)
  last = (NGRP - 1) % 2
  for b in range(G):
    drain_scatter(last * G + b)
  pltpu.sync_copy(g_hbm.at[idx_s.at[pl.ds(NCHUNK * CHUNK, REM)]],
                  rows.at[0, pl.ds(0, REM)])
  pltpu.sync_copy(rows.at[0, pl.ds(0, REM)],
                  acc.at[idx_d.at[pl.ds(NCHUNK * CHUNK, REM)]], add=True)
  plsc.subcore_barrier()
  pltpu.sync_copy(acc.at[pl.ds(s * ROWS_PER_S, ROWS_PER_S)],
                  out_hbm.at[c, pl.ds(s * ROWS_PER_S, ROWS_PER_S)])


def _sc_scatter(g, ei_flat):
  zeros = jnp.zeros((NPAD, HDIM), jnp.float32)
  return pl.kernel(
      _edge_body,
      out_type=jax.ShapeDtypeStruct((NC, NPAD, HDIM), jnp.float32),
      mesh=_SC_MESH,
      scratch_types=[
          pltpu.VMEM((E_PER_W,), jnp.int32),
          pltpu.VMEM((E_PER_W,), jnp.int32),
          pltpu.VMEM((2 * G, CHUNK, HDIM), jnp.float32),
          pltpu.VMEM_SHARED((NPAD, HDIM), jnp.float32),
          pltpu.SemaphoreType.DMA,
          pltpu.SemaphoreType.DMA,
      ],
      compiler_params=_SC_PARAMS,
  )(g, ei_flat, zeros)


def _mm_body(x_ref, wx_ref, bx_ref, wg_ref, h2_ref):
  h = jnp.dot(x_ref[...], wx_ref[...], preferred_element_type=jnp.float32)
  h = jnp.maximum(h + bx_ref[...], 0.0)
  h2_ref[...] = jnp.dot(h, wg_ref[...], preferred_element_type=jnp.float32)


def _tc_mm(x, Wx, bx, Wg):
  return pl.pallas_call(
      _mm_body,
      grid=(N // BR,),
      in_specs=[
          pl.BlockSpec((BR, NFEAT), lambda i: (i, 0)),
          pl.BlockSpec((NFEAT, HDIM), lambda i: (0, 0)),
          pl.BlockSpec((1, HDIM), lambda i: (0, 0)),
          pl.BlockSpec((HDIM, HDIM), lambda i: (0, 0)),
      ],
      out_specs=pl.BlockSpec((BR, HDIM), lambda i: (i, 0)),
      out_shape=jax.ShapeDtypeStruct((N, HDIM), jnp.float32),
  )(x, Wx, bx.reshape(1, HDIM), Wg)


def _scale_body(h2_ref, dp_ref, g_ref):
  dp = dp_ref[...]
  deg = dp[0] + dp[1] + 1.0
  dinv = lax.rsqrt(deg)[:, 0:1]
  g_ref[...] = h2_ref[...] * dinv


def _tc_scale(h2, deg_parts):
  return pl.pallas_call(
      _scale_body,
      grid=(N // BR,),
      in_specs=[
          pl.BlockSpec((BR, HDIM), lambda i: (i, 0)),
          pl.BlockSpec((NC, BR, 16), lambda i: (0, i, 0)),
      ],
      out_specs=pl.BlockSpec((BR, HDIM), lambda i: (i, 0)),
      out_shape=jax.ShapeDtypeStruct((N, HDIM), jnp.float32),
  )(h2, deg_parts)


def _back_body(raw_ref, dp_ref, g_ref, bg_ref, wz_ref, bz_ref, out_ref):
  dp = dp_ref[...]
  deg = dp[0] + dp[1] + 1.0
  dinv = lax.rsqrt(deg)[:, 0:1]
  raw = raw_ref[...]
  l1 = (raw[0] + raw[1] + g_ref[...]) * dinv + bg_ref[...]
  z = jnp.dot(l1, wz_ref[...], preferred_element_type=jnp.float32)
  out_ref[...] = jnp.maximum(z + bz_ref[...], 0.0).T


def _tc_back(raw, deg_parts, g, bg, Wz, bz):
  br = 2048
  out_t = pl.pallas_call(
      _back_body,
      grid=(NPAD // br,),
      in_specs=[
          pl.BlockSpec((NC, br, HDIM), lambda i: (0, i, 0)),
          pl.BlockSpec((NC, br, 16), lambda i: (0, i, 0)),
          pl.BlockSpec((br, HDIM), lambda i: (i, 0)),
          pl.BlockSpec((1, HDIM), lambda i: (0, 0)),
          pl.BlockSpec((HDIM, NCLASS), lambda i: (0, 0)),
          pl.BlockSpec((1, NCLASS), lambda i: (0, 0)),
      ],
      out_specs=pl.BlockSpec((NCLASS, br), lambda i: (0, i)),
      out_shape=jax.ShapeDtypeStruct((NCLASS, NPAD), jnp.float32),
  )(raw, deg_parts, g, bg.reshape(1, HDIM), Wz, bz.reshape(1, NCLASS))
  return out_t[:, :N].T


def kernel(x, edge_index, Wx, bx, Wg, bg, Wz, bz):
  ei_flat = edge_index.reshape(2 * E)
  deg_parts = _sc_degree(ei_flat)
  h2 = _tc_mm(x, Wx, bx, Wg)
  g = _tc_scale(h2, deg_parts)
  raw = _sc_scatter(g, ei_flat)
  return _tc_back(raw, deg_parts, g, bg, Wz, bz)


# speedup vs baseline: 1.0087x; 1.0087x over previous
import jax
import jax.numpy as jnp
from jax import lax
from jax.experimental import pallas as pl
from jax.experimental.pallas import tpu as pltpu
from jax.experimental.pallas import tpu_sc as plsc

N = 10000
E = 320000
NFEAT = 128
HDIM = 64
NCLASS = 40

NC = 2
NS = 16
NW = NC * NS
E_PER_W = E // NW
CHUNK = 80
NCHUNK = E_PER_W // CHUNK
G = 5
NGRP = NCHUNK // G
LAG = 16
NPAD = 10240
ROWS_PER_S = NPAD // NS
BR = 2000

_SC_MESH = plsc.VectorSubcoreMesh(
    core_axis_name="c", subcore_axis_name="s", num_cores=NC, num_subcores=NS)
_SC_PARAMS = pltpu.CompilerParams(use_tc_tiling_on_sc=False)


def _deg_body(ei_hbm, ones_hbm, zeros_hbm, out_hbm, idx_d, ones_v, acc, sem):
  c = lax.axis_index("c")
  s = lax.axis_index("s")
  pltpu.sync_copy(zeros_hbm.at[pl.ds(s * ROWS_PER_S, ROWS_PER_S)],
                  acc.at[pl.ds(s * ROWS_PER_S, ROWS_PER_S)])
  pltpu.sync_copy(ei_hbm.at[pl.ds(E + (c * NS + s) * E_PER_W, E_PER_W)], idx_d)
  pltpu.sync_copy(ones_hbm, ones_v)
  plsc.subcore_barrier()

  def drain():
    pltpu.make_async_copy(ones_hbm, ones_v, sem).wait()

  def step(j, carry):
    pltpu.async_copy(ones_v, acc.at[idx_d.at[pl.ds(j * CHUNK, CHUNK)]],
                     sem, add=True)

    @pl.when(j >= LAG)
    def _():
      drain()

    return carry

  lax.fori_loop(0, NCHUNK, step, 0)
  for _ in range(LAG):
    drain()
  plsc.subcore_barrier()
  pltpu.sync_copy(acc.at[pl.ds(s * ROWS_PER_S, ROWS_PER_S)],
                  out_hbm.at[c, pl.ds(s * ROWS_PER_S, ROWS_PER_S)])


def _sc_degree(ei_flat):
  ones = jnp.ones((CHUNK, 16), jnp.float32)
  zeros = jnp.zeros((NPAD, 16), jnp.float32)
  return pl.kernel(
      _deg_body,
      out_type=jax.ShapeDtypeStruct((NC, NPAD, 16), jnp.float32),
      mesh=_SC_MESH,
      scratch_types=[
          pltpu.VMEM((E_PER_W,), jnp.int32),
          pltpu.VMEM((CHUNK, 16), jnp.float32),
          pltpu.VMEM_SHARED((NPAD, 16), jnp.float32),
          pltpu.SemaphoreType.DMA,
      ],
      compiler_params=_SC_PARAMS,
  )(ei_flat, ones, zeros)


def _edge_body(g_hbm, ei_hbm, zeros_hbm, out_hbm,
               idx_s, idx_d, rows, acc, gsem, ssem):
  c = lax.axis_index("c")
  s = lax.axis_index("s")
  base = (c * NS + s) * E_PER_W
  pltpu.sync_copy(ei_hbm.at[pl.ds(base, E_PER_W)], idx_s)

  def fire_gather(j, slot):
    pltpu.async_copy(g_hbm.at[idx_s.at[pl.ds(j * CHUNK, CHUNK)]],
                     rows.at[slot], gsem)

  def drain_gather(slot):
    pltpu.make_async_copy(g_hbm.at[pl.ds(0, CHUNK)], rows.at[slot], gsem).wait()

  def fire_scatter(j, slot):
    pltpu.async_copy(rows.at[slot], acc.at[idx_d.at[pl.ds(j * CHUNK, CHUNK)]],
                     ssem, add=True)

  def drain_scatter(slot):
    pltpu.make_async_copy(g_hbm.at[pl.ds(0, CHUNK)], rows.at[slot], ssem).wait()

  for b in range(G):
    fire_gather(b, b)
  pltpu.sync_copy(ei_hbm.at[pl.ds(E + base, E_PER_W)], idx_d)
  pltpu.sync_copy(zeros_hbm.at[pl.ds(s * ROWS_PER_S, ROWS_PER_S)],
                  acc.at[pl.ds(s * ROWS_PER_S, ROWS_PER_S)])
  plsc.subcore_barrier()

  def body(J, carry):
    cur = lax.rem(J, 2)
    nxt = 1 - cur

    @pl.when(J >= 1)
    def _():
      for b in range(G):
        drain_scatter(nxt * G + b)

    @pl.when(J + 1 < NGRP)
    def _():
      for b in range(G):
        fire_gather((J + 1) * G + b, nxt * G + b)

    for b in range(G):
      drain_gather(cur * G + b)
    for b in range(G):
      fire_scatter(J * G + b, cur * G + b)
    return carry

  lax.fori_loop(0, NGRP, body, 0)
  last = (NGRP - 1) % 2
  for b in range(G):
    drain_scatter(last * G + b)
  plsc.subcore_barrier()
  pltpu.sync_copy(acc.at[pl.ds(s * ROWS_PER_S, ROWS_PER_S)],
                  out_hbm.at[c, pl.ds(s * ROWS_PER_S, ROWS_PER_S)])


def _sc_scatter(g, ei_flat):
  zeros = jnp.zeros((NPAD, HDIM), jnp.float32)
  return pl.kernel(
      _edge_body,
      out_type=jax.ShapeDtypeStruct((NC, NPAD, HDIM), jnp.float32),
      mesh=_SC_MESH,
      scratch_types=[
          pltpu.VMEM((E_PER_W,), jnp.int32),
          pltpu.VMEM((E_PER_W,), jnp.int32),
          pltpu.VMEM((2 * G, CHUNK, HDIM), jnp.float32),
          pltpu.VMEM_SHARED((NPAD, HDIM), jnp.float32),
          pltpu.SemaphoreType.DMA,
          pltpu.SemaphoreType.DMA,
      ],
      compiler_params=_SC_PARAMS,
  )(g, ei_flat, zeros)


def _mm_body(x_ref, wx_ref, bx_ref, wg_ref, h2_ref):
  h = jnp.dot(x_ref[...], wx_ref[...], preferred_element_type=jnp.float32)
  h = jnp.maximum(h + bx_ref[...], 0.0)
  h2_ref[...] = jnp.dot(h, wg_ref[...], preferred_element_type=jnp.float32)


def _tc_mm(x, Wx, bx, Wg):
  return pl.pallas_call(
      _mm_body,
      grid=(N // BR,),
      in_specs=[
          pl.BlockSpec((BR, NFEAT), lambda i: (i, 0)),
          pl.BlockSpec((NFEAT, HDIM), lambda i: (0, 0)),
          pl.BlockSpec((1, HDIM), lambda i: (0, 0)),
          pl.BlockSpec((HDIM, HDIM), lambda i: (0, 0)),
      ],
      out_specs=pl.BlockSpec((BR, HDIM), lambda i: (i, 0)),
      out_shape=jax.ShapeDtypeStruct((N, HDIM), jnp.float32),
  )(x, Wx, bx.reshape(1, HDIM), Wg)


def _scale_body(h2_ref, dp_ref, g_ref):
  dp = dp_ref[...]
  deg = dp[0] + dp[1] + 1.0
  dinv = lax.rsqrt(deg)[:, 0:1]
  g_ref[...] = h2_ref[...] * dinv


def _tc_scale(h2, deg_parts):
  return pl.pallas_call(
      _scale_body,
      grid=(N // BR,),
      in_specs=[
          pl.BlockSpec((BR, HDIM), lambda i: (i, 0)),
          pl.BlockSpec((NC, BR, 16), lambda i: (0, i, 0)),
      ],
      out_specs=pl.BlockSpec((BR, HDIM), lambda i: (i, 0)),
      out_shape=jax.ShapeDtypeStruct((N, HDIM), jnp.float32),
  )(h2, deg_parts)


def _back_body(raw_ref, dp_ref, g_ref, bg_ref, wz_ref, bz_ref, out_ref):
  dp = dp_ref[...]
  deg = dp[0] + dp[1] + 1.0
  dinv = lax.rsqrt(deg)[:, 0:1]
  raw = raw_ref[...]
  l1 = (raw[0] + raw[1] + g_ref[...]) * dinv + bg_ref[...]
  z = jnp.dot(l1, wz_ref[...], preferred_element_type=jnp.float32)
  out_ref[...] = jnp.maximum(z + bz_ref[...], 0.0).T


def _tc_back(raw, deg_parts, g, bg, Wz, bz):
  br = 2048
  out_t = pl.pallas_call(
      _back_body,
      grid=(NPAD // br,),
      in_specs=[
          pl.BlockSpec((NC, br, HDIM), lambda i: (0, i, 0)),
          pl.BlockSpec((NC, br, 16), lambda i: (0, i, 0)),
          pl.BlockSpec((br, HDIM), lambda i: (i, 0)),
          pl.BlockSpec((1, HDIM), lambda i: (0, 0)),
          pl.BlockSpec((HDIM, NCLASS), lambda i: (0, 0)),
          pl.BlockSpec((1, NCLASS), lambda i: (0, 0)),
      ],
      out_specs=pl.BlockSpec((NCLASS, br), lambda i: (0, i)),
      out_shape=jax.ShapeDtypeStruct((NCLASS, NPAD), jnp.float32),
  )(raw, deg_parts, g, bg.reshape(1, HDIM), Wz, bz.reshape(1, NCLASS))
  return out_t[:, :N].T


def kernel(x, edge_index, Wx, bx, Wg, bg, Wz, bz):
  ei_flat = edge_index.reshape(2 * E)
  deg_parts = _sc_degree(ei_flat)
  h2 = _tc_mm(x, Wx, bx, Wg)
  g = _tc_scale(h2, deg_parts)
  raw = _sc_scatter(g, ei_flat)
  return _tc_back(raw, deg_parts, g, bg, Wz, bz)
